# baseline (speedup 1.0000x reference)
import jax
import jax.numpy as jnp
from jax import lax
from jax.experimental import pallas as pl
from jax.experimental.pallas import tpu as pltpu
from jax.experimental.pallas import tpu_sc as plsc

NODES = 10000
PAD = 10240
DIM = 512
KNB = 8

BQ = 512
CW = 512
GG = 128
BIGF = 1e30
BIGF2 = 2e30
_DN = (((1,), (0,)), ((), ()))


def _d2_chunk(qb, sqq, posT_ref, posTb_ref, o):
    qxb, qyb, qzb = qb
    cxb = posTb_ref[0:1, pl.ds(o, CW)].astype(jnp.float32)
    cyb = posTb_ref[1:2, pl.ds(o, CW)].astype(jnp.float32)
    czb = posTb_ref[2:3, pl.ds(o, CW)].astype(jnp.float32)
    gram = (qxb * cxb + qyb * cyb) + qzb * czb
    cx = posT_ref[0:1, pl.ds(o, CW)]
    cy = posT_ref[1:2, pl.ds(o, CW)]
    cz = posT_ref[2:3, pl.ds(o, CW)]
    sqc = (cx * cx + cy * cy) + cz * cz
    d2 = (sqq - 2.0 * gram) + sqc
    col = lax.broadcasted_iota(jnp.int32, (1, CW), 1) + o
    d2 = jnp.where(col >= NODES, BIGF, d2)
    return d2, col


def _knn_body(posq_ref, posT_ref, posTb_ref, idx_ref):
    qx = posq_ref[:, 0:1]
    qy = posq_ref[:, 1:2]
    qz = posq_ref[:, 2:3]
    sqq = (qx * qx + qy * qy) + qz * qz

    def _b(t):
        return t.astype(jnp.bfloat16).astype(jnp.float32)

    qb = (_b(qx), _b(qy), _b(qz))

    nch = PAD // CW

    def build_chunk(ci, carry):
        s1, s2, s3, s4, i1, i2, i3 = carry
        o = ci * CW
        d2, col = _d2_chunk(qb, sqq, posT_ref, posTb_ref, o)
        for g in range(CW // GG):
            v = d2[:, g * GG:(g + 1) * GG]
            cv = col[:, g * GG:(g + 1) * GG]
            c1 = v < s1
            c2 = v < s2
            c3 = v < s3
            c4 = v < s4
            s4 = jnp.where(c4, jnp.where(c3, s3, v), s4)
            s3 = jnp.where(c3, jnp.where(c2, s2, v), s3)
            i3 = jnp.where(c3, jnp.where(c2, i2, cv), i3)
            s2 = jnp.where(c2, jnp.where(c1, s1, v), s2)
            i2 = jnp.where(c2, jnp.where(c1, i1, cv), i2)
            s1 = jnp.where(c1, v, s1)
            i1 = jnp.where(c1, cv, i1)
        return (s1, s2, s3, s4, i1, i2, i3)

    f0 = jnp.full((BQ, GG), BIGF2, jnp.float32)
    j0 = jnp.full((BQ, GG), PAD, jnp.int32)
    s1, s2, s3, s4, i1, i2, i3 = lax.fori_loop(
        0, nch, build_chunk, (f0, f0, f0, f0, j0, j0, j0))

    V = jnp.concatenate([s1, s2, s3], axis=1)
    I = jnp.concatenate([i1, i2, i3], axis=1)
    m8 = None
    for k in range(KNB):
        m8 = jnp.min(V, axis=1, keepdims=True)
        sel = V == m8
        i8 = jnp.min(jnp.where(sel, I, PAD), axis=1, keepdims=True)
        idx_ref[:, k:k + 1] = i8
        V = jnp.where(sel & (I == i8), BIGF2, V)

    ok = jnp.all(s4 > m8)

    @pl.when(jnp.logical_not(ok))
    def _fallback():
        pm = jnp.full((BQ, 1), -1e38, jnp.float32)
        pi = jnp.full((BQ, 1), -1, jnp.int32)
        for k in range(KNB):
            def scan(ci, carry, pm=pm, pi=pi):
                m, idx = carry
                o = ci * CW
                v, col = _d2_chunk(qb, sqq, posT_ref, posTb_ref, o)
                keep = (v > pm) | ((v == pm) & (col > pi))
                vv = jnp.where(keep, v, BIGF2)
                cm = jnp.min(vv, axis=1, keepdims=True)
                cc = jnp.min(jnp.where(vv == cm, col, PAD),
                             axis=1, keepdims=True)
                better = (cm < m) | ((cm == m) & (cc < idx))
                return (jnp.where(better, cm, m), jnp.where(better, cc, idx))

            mk, ik = lax.fori_loop(
                0, nch, scan,
                (jnp.full((BQ, 1), 1e38, jnp.float32),
                 jnp.full((BQ, 1), PAD, jnp.int32)))
            idx_ref[:, k:k + 1] = ik
            pm, pi = mk, ik


def _knn(pos_pad, posT, posTb):
    return pl.pallas_call(
        _knn_body,
        grid=(PAD // BQ,),
        in_specs=[
            pl.BlockSpec((BQ, 8), lambda i: (i, 0)),
            pl.BlockSpec((8, PAD), lambda i: (0, 0)),
            pl.BlockSpec((8, PAD), lambda i: (0, 0)),
        ],
        out_specs=pl.BlockSpec((BQ, KNB), lambda i: (i, 0)),
        out_shape=jax.ShapeDtypeStruct((PAD, KNB), jnp.int32),
    )(pos_pad, posT, posTb)


NW = 32
NPW = PAD // NW
CN = 8
NCH_GM = NPW // CN

def _sc_mesh():
    return plsc.VectorSubcoreMesh(core_axis_name="c", subcore_axis_name="s",
                                  num_cores=2, num_subcores=16)


def _gm_body(h_hbm, idx_hbm, out_hbm, idx_v, rows0, rows1, acc_v, sem0, sem1):
    c = lax.axis_index("c")
    s = lax.axis_index("s")
    wid = s * 2 + c
    base = wid * NPW
    pltpu.sync_copy(idx_hbm.at[pl.ds(base * KNB, NPW * KNB)], idx_v)

    def start(ci, buf, sem):
        sl = idx_v.at[pl.ds(ci * CN * KNB, CN * KNB)]
        pltpu.async_copy(h_hbm.at[sl], buf, sem)

    def wait(ci, buf, sem):
        sl = idx_v.at[pl.ds(ci * CN * KNB, CN * KNB)]
        pltpu.make_async_copy(h_hbm.at[sl], buf, sem).wait()

    def accum_store(ci, buf):
        for n in range(CN):
            for j in range(DIM // 16):
                sl = pl.ds(j * 16, 16)
                acc = buf[n * KNB + 0, sl]
                for kk in range(1, KNB):
                    acc = acc + buf[n * KNB + kk, sl]
                acc_v[n, sl] = acc * (1.0 / KNB)
        pltpu.sync_copy(acc_v, out_hbm.at[pl.ds(base + ci * CN, CN)])

    start(0, rows0, sem0)

    def pair(p, carry):
        c0 = 2 * p
        start(c0 + 1, rows1, sem1)
        wait(c0, rows0, sem0)
        accum_store(c0, rows0)

        @pl.when(c0 + 2 < NCH_GM)
        def _():
            start(c0 + 2, rows0, sem0)

        wait(c0 + 1, rows1, sem1)
        accum_store(c0 + 1, rows1)
        return carry

    lax.fori_loop(0, NCH_GM // 2, pair, 0)


def _gather_mean(hp, idx_flat):
    return pl.kernel(
        _gm_body,
        out_type=jax.ShapeDtypeStruct((PAD, DIM), jnp.float32),
        mesh=_sc_mesh(),
        scratch_types=[
            pltpu.VMEM((NPW * KNB,), jnp.int32),
            pltpu.VMEM((CN * KNB, DIM), jnp.float32),
            pltpu.VMEM((CN * KNB, DIM), jnp.float32),
            pltpu.VMEM((CN, DIM), jnp.float32),
            pltpu.SemaphoreType.DMA,
            pltpu.SemaphoreType.DMA,
        ],
    )(hp, idx_flat)


BM = 512
_DNT = (((1,), (1,)), ((), ()))


def _layer_pre_body(h_ref, wr_ref, b_ref, o_ref):
    acc = lax.dot_general(h_ref[...], wr_ref[...], _DNT,
                          preferred_element_type=jnp.float32)
    o_ref[...] = acc + b_ref[...]


def _layer_pre(hp, Wr, bl):
    return pl.pallas_call(
        _layer_pre_body,
        grid=(PAD // BM,),
        in_specs=[
            pl.BlockSpec((BM, DIM), lambda i: (i, 0)),
            pl.BlockSpec((DIM, DIM), lambda i: (0, 0)),
            pl.BlockSpec((1, DIM), lambda i: (0, 0)),
        ],
        out_specs=pl.BlockSpec((BM, DIM), lambda i: (i, 0)),
        out_shape=jax.ShapeDtypeStruct((PAD, DIM), jnp.float32),
    )(hp, Wr, bl.reshape(1, DIM))


def _layer_post_body(a_ref, wl_ref, t_ref, o_ref):
    acc = lax.dot_general(a_ref[...], wl_ref[...], _DNT,
                          preferred_element_type=jnp.float32)
    o_ref[...] = jnp.maximum(acc + t_ref[...], 0.0)


def _layer_post(aggr, Wl, t):
    return pl.pallas_call(
        _layer_post_body,
        grid=(PAD // BM,),
        in_specs=[
            pl.BlockSpec((BM, DIM), lambda i: (i, 0)),
            pl.BlockSpec((DIM, DIM), lambda i: (0, 0)),
            pl.BlockSpec((BM, DIM), lambda i: (i, 0)),
        ],
        out_specs=pl.BlockSpec((BM, DIM), lambda i: (i, 0)),
        out_shape=jax.ShapeDtypeStruct((PAD, DIM), jnp.float32),
    )(aggr, Wl, t)


def kernel(h_obs, h_bg, pos_obs, Wl0, bl0, Wr0, Wl1, bl1, Wr1):
    pos_pad = jnp.zeros((PAD, 8), jnp.float32).at[:NODES, :3].set(pos_obs)
    posT = pos_pad.T
    posTb = posT.astype(jnp.bfloat16)
    idx = _knn(pos_pad, posT, posTb)
    idx_flat = idx.reshape(-1)
    hp = jnp.concatenate(
        [h_obs, jnp.zeros((PAD - NODES, DIM), h_obs.dtype)], axis=0)
    aggr0 = _gather_mean(hp, idx_flat)
    t0 = _layer_pre(hp, Wr0, bl0)
    h1 = _layer_post(aggr0, Wl0, t0)
    aggr1 = _gather_mean(h1, idx_flat)
    t1 = _layer_pre(h1, Wr1, bl1)
    h2 = _layer_post(aggr1, Wl1, t1)
    return h2[:NODES]


# speedup vs baseline: 1.0108x; 1.0108x over previous
import jax
import jax.numpy as jnp
from jax import lax
from jax.experimental import pallas as pl
from jax.experimental.pallas import tpu as pltpu
from jax.experimental.pallas import tpu_sc as plsc

NODES = 10000
PAD = 10240
DIM = 512
KNB = 8

BQ = 256
CW = 512
GG = 128
BIGF = 1e30
BIGF2 = 2e30
_DN = (((1,), (0,)), ((), ()))


def _d2_chunk(qb, sqq, posT_ref, posTb_ref, o):
    qxb, qyb, qzb = qb
    cxb = posTb_ref[0:1, pl.ds(o, CW)].astype(jnp.float32)
    cyb = posTb_ref[1:2, pl.ds(o, CW)].astype(jnp.float32)
    czb = posTb_ref[2:3, pl.ds(o, CW)].astype(jnp.float32)
    gram = (qxb * cxb + qyb * cyb) + qzb * czb
    cx = posT_ref[0:1, pl.ds(o, CW)]
    cy = posT_ref[1:2, pl.ds(o, CW)]
    cz = posT_ref[2:3, pl.ds(o, CW)]
    sqc = (cx * cx + cy * cy) + cz * cz
    d2 = (sqq - 2.0 * gram) + sqc
    col = lax.broadcasted_iota(jnp.int32, (1, CW), 1) + o
    d2 = jnp.where(col >= NODES, BIGF, d2)
    return d2, col


def _knn_body(posq_ref, posT_ref, posTb_ref, idx_ref):
    qx = posq_ref[:, 0:1]
    qy = posq_ref[:, 1:2]
    qz = posq_ref[:, 2:3]
    sqq = (qx * qx + qy * qy) + qz * qz

    def _b(t):
        return t.astype(jnp.bfloat16).astype(jnp.float32)

    qb = (_b(qx), _b(qy), _b(qz))

    nch = PAD // CW

    def build_chunk(ci, carry):
        s1, s2, s3, s4, i1, i2, i3 = carry
        o = ci * CW
        d2, col = _d2_chunk(qb, sqq, posT_ref, posTb_ref, o)
        for g in range(CW // GG):
            v = d2[:, g * GG:(g + 1) * GG]
            cv = col[:, g * GG:(g + 1) * GG]
            c1 = v < s1
            c2 = v < s2
            c3 = v < s3
            c4 = v < s4
            s4 = jnp.where(c4, jnp.where(c3, s3, v), s4)
            s3 = jnp.where(c3, jnp.where(c2, s2, v), s3)
            i3 = jnp.where(c3, jnp.where(c2, i2, cv), i3)
            s2 = jnp.where(c2, jnp.where(c1, s1, v), s2)
            i2 = jnp.where(c2, jnp.where(c1, i1, cv), i2)
            s1 = jnp.where(c1, v, s1)
            i1 = jnp.where(c1, cv, i1)
        return (s1, s2, s3, s4, i1, i2, i3)

    f0 = jnp.full((BQ, GG), BIGF2, jnp.float32)
    j0 = jnp.full((BQ, GG), PAD, jnp.int32)
    s1, s2, s3, s4, i1, i2, i3 = lax.fori_loop(
        0, nch, build_chunk, (f0, f0, f0, f0, j0, j0, j0))

    V = jnp.concatenate([s1, s2, s3], axis=1)
    I = jnp.concatenate([i1, i2, i3], axis=1)
    m8 = None
    for k in range(KNB):
        m8 = jnp.min(V, axis=1, keepdims=True)
        sel = V == m8
        i8 = jnp.min(jnp.where(sel, I, PAD), axis=1, keepdims=True)
        idx_ref[:, k:k + 1] = i8
        V = jnp.where(sel & (I == i8), BIGF2, V)

    ok = jnp.all(s4 > m8)

    @pl.when(jnp.logical_not(ok))
    def _fallback():
        pm = jnp.full((BQ, 1), -1e38, jnp.float32)
        pi = jnp.full((BQ, 1), -1, jnp.int32)
        for k in range(KNB):
            def scan(ci, carry, pm=pm, pi=pi):
                m, idx = carry
                o = ci * CW
                v, col = _d2_chunk(qb, sqq, posT_ref, posTb_ref, o)
                keep = (v > pm) | ((v == pm) & (col > pi))
                vv = jnp.where(keep, v, BIGF2)
                cm = jnp.min(vv, axis=1, keepdims=True)
                cc = jnp.min(jnp.where(vv == cm, col, PAD),
                             axis=1, keepdims=True)
                better = (cm < m) | ((cm == m) & (cc < idx))
                return (jnp.where(better, cm, m), jnp.where(better, cc, idx))

            mk, ik = lax.fori_loop(
                0, nch, scan,
                (jnp.full((BQ, 1), 1e38, jnp.float32),
                 jnp.full((BQ, 1), PAD, jnp.int32)))
            idx_ref[:, k:k + 1] = ik
            pm, pi = mk, ik


def _knn(pos_pad, posT, posTb):
    return pl.pallas_call(
        _knn_body,
        grid=(PAD // BQ,),
        in_specs=[
            pl.BlockSpec((BQ, 8), lambda i: (i, 0)),
            pl.BlockSpec((8, PAD), lambda i: (0, 0)),
            pl.BlockSpec((8, PAD), lambda i: (0, 0)),
        ],
        out_specs=pl.BlockSpec((BQ, KNB), lambda i: (i, 0)),
        out_shape=jax.ShapeDtypeStruct((PAD, KNB), jnp.int32),
    )(pos_pad, posT, posTb)


NW = 32
NPW = PAD // NW
CN = 8
NCH_GM = NPW // CN

def _sc_mesh():
    return plsc.VectorSubcoreMesh(core_axis_name="c", subcore_axis_name="s",
                                  num_cores=2, num_subcores=16)


def _gm_body(h_hbm, idx_hbm, out_hbm, idx_v, rows0, rows1, acc_v, sem0, sem1):
    c = lax.axis_index("c")
    s = lax.axis_index("s")
    wid = s * 2 + c
    base = wid * NPW
    pltpu.sync_copy(idx_hbm.at[pl.ds(base * KNB, NPW * KNB)], idx_v)

    def start(ci, buf, sem):
        sl = idx_v.at[pl.ds(ci * CN * KNB, CN * KNB)]
        pltpu.async_copy(h_hbm.at[sl], buf, sem)

    def wait(ci, buf, sem):
        sl = idx_v.at[pl.ds(ci * CN * KNB, CN * KNB)]
        pltpu.make_async_copy(h_hbm.at[sl], buf, sem).wait()

    def accum_store(ci, buf):
        for n in range(CN):
            for j in range(DIM // 16):
                sl = pl.ds(j * 16, 16)
                acc = buf[n * KNB + 0, sl]
                for kk in range(1, KNB):
                    acc = acc + buf[n * KNB + kk, sl]
                acc_v[n, sl] = acc * (1.0 / KNB)
        pltpu.sync_copy(acc_v, out_hbm.at[pl.ds(base + ci * CN, CN)])

    start(0, rows0, sem0)

    def pair(p, carry):
        c0 = 2 * p
        start(c0 + 1, rows1, sem1)
        wait(c0, rows0, sem0)
        accum_store(c0, rows0)

        @pl.when(c0 + 2 < NCH_GM)
        def _():
            start(c0 + 2, rows0, sem0)

        wait(c0 + 1, rows1, sem1)
        accum_store(c0 + 1, rows1)
        return carry

    lax.fori_loop(0, NCH_GM // 2, pair, 0)


def _gather_mean(hp, idx_flat):
    return pl.kernel(
        _gm_body,
        out_type=jax.ShapeDtypeStruct((PAD, DIM), jnp.float32),
        mesh=_sc_mesh(),
        scratch_types=[
            pltpu.VMEM((NPW * KNB,), jnp.int32),
            pltpu.VMEM((CN * KNB, DIM), jnp.float32),
            pltpu.VMEM((CN * KNB, DIM), jnp.float32),
            pltpu.VMEM((CN, DIM), jnp.float32),
            pltpu.SemaphoreType.DMA,
            pltpu.SemaphoreType.DMA,
        ],
    )(hp, idx_flat)


BM = 512
_DNT = (((1,), (1,)), ((), ()))


def _layer_pre_body(h_ref, wr_ref, b_ref, o_ref):
    acc = lax.dot_general(h_ref[...], wr_ref[...], _DNT,
                          preferred_element_type=jnp.float32)
    o_ref[...] = acc + b_ref[...]


def _layer_pre(hp, Wr, bl):
    return pl.pallas_call(
        _layer_pre_body,
        grid=(PAD // BM,),
        in_specs=[
            pl.BlockSpec((BM, DIM), lambda i: (i, 0)),
            pl.BlockSpec((DIM, DIM), lambda i: (0, 0)),
            pl.BlockSpec((1, DIM), lambda i: (0, 0)),
        ],
        out_specs=pl.BlockSpec((BM, DIM), lambda i: (i, 0)),
        out_shape=jax.ShapeDtypeStruct((PAD, DIM), jnp.float32),
    )(hp, Wr, bl.reshape(1, DIM))


def _layer_post_body(a_ref, wl_ref, t_ref, o_ref):
    acc = lax.dot_general(a_ref[...], wl_ref[...], _DNT,
                          preferred_element_type=jnp.float32)
    o_ref[...] = jnp.maximum(acc + t_ref[...], 0.0)


def _layer_post(aggr, Wl, t):
    return pl.pallas_call(
        _layer_post_body,
        grid=(PAD // BM,),
        in_specs=[
            pl.BlockSpec((BM, DIM), lambda i: (i, 0)),
            pl.BlockSpec((DIM, DIM), lambda i: (0, 0)),
            pl.BlockSpec((BM, DIM), lambda i: (i, 0)),
        ],
        out_specs=pl.BlockSpec((BM, DIM), lambda i: (i, 0)),
        out_shape=jax.ShapeDtypeStruct((PAD, DIM), jnp.float32),
    )(aggr, Wl, t)


def kernel(h_obs, h_bg, pos_obs, Wl0, bl0, Wr0, Wl1, bl1, Wr1):
    pos_pad = jnp.zeros((PAD, 8), jnp.float32).at[:NODES, :3].set(pos_obs)
    posT = pos_pad.T
    posTb = posT.astype(jnp.bfloat16)
    idx = _knn(pos_pad, posT, posTb)
    idx_flat = idx.reshape(-1)
    hp = jnp.concatenate(
        [h_obs, jnp.zeros((PAD - NODES, DIM), h_obs.dtype)], axis=0)
    aggr0 = _gather_mean(hp, idx_flat)
    t0 = _layer_pre(hp, Wr0, bl0)
    h1 = _layer_post(aggr0, Wl0, t0)
    aggr1 = _gather_mean(h1, idx_flat)
    t1 = _layer_pre(h1, Wr1, bl1)
    h2 = _layer_post(aggr1, Wl1, t1)
    return h2[:NODES]
